# baseline (speedup 1.0000x reference)
import functools

import jax
import jax.numpy as jnp
from jax import lax
from jax.experimental import pallas as pl
from jax.experimental.pallas import tpu as pltpu
from jax.experimental.pallas import tpu_sc as plsc

N_IDX = 6_400_000
DIM = 16
LANES = 128
PAD_SUBROWS = N_IDX * (LANES // DIM)
NUM_CORES = 2
NUM_SUBCORES = 16
NW = NUM_CORES * NUM_SUBCORES
PER_W = N_IDX // NW
CHUNK = 2_000
N_CHUNKS = PER_W // CHUNK
GROUPS = CHUNK // 16


def _sc_lookup(table_t, idx):
    mesh = plsc.VectorSubcoreMesh(core_axis_name="c", subcore_axis_name="s")

    @functools.partial(
        pl.kernel,
        mesh=mesh,
        out_type=jax.ShapeDtypeStruct((PAD_SUBROWS, DIM), jnp.float32),
        compiler_params=pltpu.CompilerParams(
            use_tc_tiling_on_sc=False, needs_layout_passes=False
        ),
        scratch_types=[
            pltpu.VMEM((DIM, DIM), jnp.float32),
            pltpu.VMEM((CHUNK,), jnp.int32),
            pltpu.VMEM((CHUNK,), jnp.int32),
            pltpu.VMEM((CHUNK,), jnp.int32),
            pltpu.VMEM((CHUNK,), jnp.int32),
            pltpu.VMEM((CHUNK, DIM), jnp.float32),
            pltpu.VMEM((CHUNK, DIM), jnp.float32),
            pltpu.SemaphoreType.DMA,
            pltpu.SemaphoreType.DMA,
            pltpu.SemaphoreType.DMA,
            pltpu.SemaphoreType.DMA,
        ],
    )
    def body(tt_hbm, idx_hbm, out_hbm, tt_v, idx_v0, idx_v1, rid_v0, rid_v1,
             rows_v0, rows_v1, sem_in0, sem_in1, sem_out0, sem_out1):
        wid = lax.axis_index("s") * NUM_CORES + lax.axis_index("c")
        base = wid * PER_W

        pltpu.sync_copy(tt_hbm, tt_v)
        tcols = [tt_v[d, :] for d in range(DIM)]

        iota16 = lax.iota(jnp.int32, 16)
        lanevec = (iota16 // 8) * 64 + (iota16 % 8) * 8
        idx_bufs = (idx_v0, idx_v1)
        rid_bufs = (rid_v0, rid_v1)
        rows_bufs = (rows_v0, rows_v1)
        sin = (sem_in0, sem_in1)
        sout = (sem_out0, sem_out1)

        pltpu.async_copy(idx_hbm.at[pl.ds(base, CHUNK)], idx_v0, sem_in0)
        pltpu.async_copy(idx_hbm.at[pl.ds(base + CHUNK, CHUNK)], idx_v1,
                         sem_in1)

        def outer(t, carry):
            for b in range(2):
                g = t * 2 + b
                start = base + g * CHUNK

                @pl.when(g >= 2)
                def _wait_out():
                    pltpu.make_async_copy(
                        rows_bufs[b], out_hbm.at[rid_bufs[b]], sout[b]
                    ).wait()

                pltpu.make_async_copy(
                    idx_hbm.at[pl.ds(base, CHUNK)], idx_bufs[b], sin[b]
                ).wait()

                def group(j, c):
                    idxv = idx_bufs[b][pl.ds(j * 16, 16)]
                    rowv = j * 16 + iota16
                    for d in range(DIM):
                        col = jnp.take_along_axis(
                            tcols[d], idxv, axis=0, mode="promise_in_bounds"
                        )
                        plsc.store_scatter(
                            rows_bufs[b],
                            [rowv, jnp.full((16,), d, jnp.int32)],
                            col,
                        )
                    rid_bufs[b][pl.ds(j * 16, 16)] = (
                        (start + j * 16) * 8 + lanevec
                    )
                    return c

                lax.fori_loop(0, GROUPS, group, 0)

                pltpu.async_copy(
                    rows_bufs[b], out_hbm.at[rid_bufs[b]], sout[b]
                )

                @pl.when(g + 2 < N_CHUNKS)
                def _prefetch():
                    pltpu.async_copy(
                        idx_hbm.at[pl.ds(start + 2 * CHUNK, CHUNK)],
                        idx_bufs[b],
                        sin[b],
                    )
            return carry

        lax.fori_loop(0, N_CHUNKS // 2, outer, 0)

        pltpu.make_async_copy(
            rows_v0, out_hbm.at[rid_v0], sem_out0
        ).wait()
        pltpu.make_async_copy(
            rows_v1, out_hbm.at[rid_v1], sem_out1
        ).wait()

    return body(table_t, idx)


def kernel(type_indices, embedding_table):
    idx = type_indices.astype(jnp.int32)
    table_t = jnp.zeros((DIM, DIM), jnp.float32)
    table_t = table_t.at[:, : embedding_table.shape[0]].set(embedding_table.T)
    padded = _sc_lookup(table_t, idx)
    return padded.reshape(N_IDX // 8, 8, LANES)[:, :, :DIM].reshape(
        N_IDX, DIM
    )


# speedup vs baseline: 16.5599x; 16.5599x over previous
import functools

import jax
import jax.numpy as jnp
from jax import lax
from jax.experimental import pallas as pl
from jax.experimental.pallas import tpu as pltpu
from jax.experimental.pallas import tpu_sc as plsc

N_IDX = 6_400_000
DIM = 16
LANES = 128
NUM_CORES = 2
NUM_SUBCORES = 16
NW = NUM_CORES * NUM_SUBCORES
ITILES = N_IDX // LANES
CHUNK_IT = 16
CHUNK = CHUNK_IT * LANES
N_CHUNKS = ITILES // CHUNK_IT
STEPS = -(-N_CHUNKS // NW)
GROUPS = CHUNK // 16
TILE_W = 8 * LANES
BUF_W = CHUNK_IT * TILE_W


def _sc_lookup(table_t, idx):
    mesh = plsc.VectorSubcoreMesh(core_axis_name="c", subcore_axis_name="s")

    @functools.partial(
        pl.kernel,
        mesh=mesh,
        out_type=jax.ShapeDtypeStruct((N_IDX * DIM,), jnp.float32),
        compiler_params=pltpu.CompilerParams(
            use_tc_tiling_on_sc=False, needs_layout_passes=False
        ),
        scratch_types=[
            pltpu.VMEM((DIM, DIM), jnp.float32),
            pltpu.VMEM((CHUNK,), jnp.int32),
            pltpu.VMEM((CHUNK,), jnp.int32),
            pltpu.VMEM((2, BUF_W), jnp.float32),
            pltpu.VMEM((2, BUF_W), jnp.float32),
            pltpu.SemaphoreType.DMA,
            pltpu.SemaphoreType.DMA,
            pltpu.SemaphoreType.DMA,
            pltpu.SemaphoreType.DMA,
        ],
    )
    def body(tt_hbm, idx_hbm, out_hbm, tt_v, idx_v0, idx_v1, rows_v0,
             rows_v1, sem_in0, sem_in1, sem_out0, sem_out1):
        wid = lax.axis_index("s") * NUM_CORES + lax.axis_index("c")

        pltpu.sync_copy(tt_hbm, tt_v)
        tcols = [tt_v[d, :] for d in range(DIM)]

        idx_bufs = (idx_v0, idx_v1)
        rows_bufs = (rows_v0, rows_v1)
        sin = (sem_in0, sem_in1)
        sout = (sem_out0, sem_out1)

        def chunk_id(m):
            return jnp.minimum(wid + NW * m, N_CHUNKS - 1)

        pltpu.async_copy(
            idx_hbm.at[pl.ds(chunk_id(0) * CHUNK, CHUNK)], idx_v0, sem_in0
        )
        pltpu.async_copy(
            idx_hbm.at[pl.ds(chunk_id(1) * CHUNK, CHUNK)], idx_v1, sem_in1
        )

        def outer(t, carry):
            for b in range(2):
                m = t * 2 + b
                c = chunk_id(m)

                @pl.when(m >= 2)
                def _wait_out():
                    for dt in range(2):
                        pltpu.make_async_copy(
                            rows_bufs[b].at[dt],
                            out_hbm.at[pl.ds(dt * ITILES * TILE_W, BUF_W)],
                            sout[b],
                        ).wait()

                pltpu.make_async_copy(
                    idx_hbm.at[pl.ds(0, CHUNK)], idx_bufs[b], sin[b]
                ).wait()

                def group(j, carry2):
                    idxv = idx_bufs[b][pl.ds(j * 16, 16)]
                    off = (j // 8) * TILE_W + (j % 8) * 16
                    for d in range(DIM):
                        col = jnp.take_along_axis(
                            tcols[d], idxv, axis=0, mode="promise_in_bounds"
                        )
                        rows_bufs[b][
                            d // 8, pl.ds(off + (d % 8) * LANES, 16)
                        ] = col
                    return carry2

                lax.fori_loop(0, GROUPS, group, 0)

                for dt in range(2):
                    pltpu.async_copy(
                        rows_bufs[b].at[dt],
                        out_hbm.at[
                            pl.ds((dt * ITILES + c * CHUNK_IT) * TILE_W,
                                  BUF_W)
                        ],
                        sout[b],
                    )

                @pl.when(m + 2 < STEPS)
                def _prefetch():
                    pltpu.async_copy(
                        idx_hbm.at[pl.ds(chunk_id(m + 2) * CHUNK, CHUNK)],
                        idx_bufs[b],
                        sin[b],
                    )
            return carry

        lax.fori_loop(0, STEPS // 2, outer, 0)

        for b in range(2):
            for dt in range(2):
                pltpu.make_async_copy(
                    rows_bufs[b].at[dt],
                    out_hbm.at[pl.ds(dt * ITILES * TILE_W, BUF_W)],
                    sout[b],
                ).wait()

    return body(table_t, idx)


def kernel(type_indices, embedding_table):
    idx = type_indices.astype(jnp.int32)
    table_t = jnp.zeros((DIM, DIM), jnp.float32)
    table_t = table_t.at[:, : embedding_table.shape[0]].set(embedding_table.T)
    flat = _sc_lookup(table_t, idx)
    r = flat.reshape(2, ITILES, 8, LANES)
    return r.transpose(1, 3, 0, 2).reshape(N_IDX, DIM)
